# baseline (speedup 1.0000x reference)
import jax
import jax.numpy as jnp
from jax import lax
from jax.experimental import pallas as pl
from jax.experimental.pallas import tpu as pltpu

_T, _D, _H, _E = 2048, 1024, 2048, 8
_BT = 128
_NTT = _T // _BT
_NEW = _NTT + _E - 1
_BTC = 256
_NCT = _T // _BTC
_NG = _NEW + _NCT
_S = 64


def _router_body(x_ref, wg_ref, bg_ref, wa_ref, ba_ref,
                 d_ref, p_ref, coef_ref, b0_ref, loss_ref, xs_ref):
    x = x_ref[...]
    logits = jnp.dot(x, wg_ref[...], preferred_element_type=jnp.float32)
    logits = logits + bg_ref[...]
    m = jnp.max(logits, axis=1, keepdims=True)
    ex = jnp.exp(logits - m)
    probs = ex / jnp.sum(ex, axis=1, keepdims=True)
    iota_e = lax.broadcasted_iota(jnp.int32, (_T, _E), 1)
    pmax = jnp.max(probs, axis=1, keepdims=True)
    idx = jnp.min(jnp.where(probs == pmax, iota_e, _E), axis=1, keepdims=True)
    disp = (iota_e == idx).astype(jnp.float32)
    gate = jnp.sum(probs * disp, axis=1, keepdims=True)

    rr = lax.broadcasted_iota(jnp.int32, (_T, _T), 0)
    cc = lax.broadcasted_iota(jnp.int32, (_T, _T), 1)
    ltri = (rr >= cc).astype(jnp.bfloat16)
    cum = jnp.dot(ltri, disp.astype(jnp.bfloat16),
                  preferred_element_type=jnp.float32)
    cnt = jnp.sum(disp, axis=0, keepdims=True)
    rank = jnp.sum(cum * disp, axis=1, keepdims=True) - 1.0
    eE_r = lax.broadcasted_iota(jnp.int32, (_E, _E), 0)
    eE_c = lax.broadcasted_iota(jnp.int32, (_E, _E), 1)
    excl = jnp.sum(jnp.transpose(cnt) * (eE_r < eE_c).astype(jnp.float32),
                   axis=0, keepdims=True)
    off_tok = jnp.sum(disp * excl, axis=1, keepdims=True)
    d_f = off_tok + rank
    d_rowf = jnp.transpose(d_f)
    d_ref[...] = d_f.astype(jnp.int32)

    x_bf = x.astype(jnp.bfloat16)
    for j in range(_NTT):
        s0 = j * _BT
        slot_col = s0 + lax.broadcasted_iota(jnp.int32, (_BT, 1), 0)
        a2 = (slot_col.astype(jnp.float32) == d_rowf).astype(jnp.bfloat16)
        xt = jnp.dot(a2, x_bf, preferred_element_type=jnp.float32)
        xs_ref[pl.ds(s0, _BT), :] = xt.astype(jnp.bfloat16)

    sump = jnp.sum(probs, axis=0, keepdims=True)
    loss_ref[...] = (_E / (_T * _T)) * jnp.sum(cnt * sump, keepdims=True).reshape(1, 1)

    bl = jnp.dot(x, wa_ref[...], preferred_element_type=jnp.float32) + ba_ref[...]
    bm = jnp.max(bl, axis=1, keepdims=True)
    be = jnp.exp(bl - bm)
    bal = be / jnp.sum(be, axis=1, keepdims=True)
    b0_ref[...] = bal[:, 0:1]
    coef_ref[...] = gate * bal[:, 1:2]

    jj = lax.broadcasted_iota(jnp.int32, (_NTT, _E), 0).astype(jnp.float32)
    ee = lax.broadcasted_iota(jnp.int32, (_NTT, _E), 1)
    lo_e = jnp.broadcast_to(excl, (_NTT, _E))
    hi_e = jnp.broadcast_to(excl + cnt, (_NTT, _E))
    t_lo = jj * _BT
    t_hi = t_lo + _BT
    ov = (lo_e < t_hi) & (hi_e > t_lo)
    ovf = ov.astype(jnp.float32)
    in_row = jnp.dot(ovf, (eE_r <= eE_c).astype(jnp.float32),
                     preferred_element_type=jnp.float32)
    rowsum = jnp.sum(ovf, axis=1, keepdims=True)
    tt_r = lax.broadcasted_iota(jnp.int32, (_NTT, _NTT), 0)
    tt_c = lax.broadcasted_iota(jnp.int32, (_NTT, _NTT), 1)
    rowpref = jnp.dot((tt_r > tt_c).astype(jnp.float32), rowsum,
                      preferred_element_type=jnp.float32)
    pos = rowpref + in_row - 1.0
    w_start = jnp.maximum(lo_e, t_lo)
    w_end = jnp.minimum(hi_e, t_hi)

    tiles, exps, sts, ens, typs, outs = [], [], [], [], [], []
    for w in range(_NEW):
        mw = ((pos == float(w)) & ov).astype(jnp.float32)
        hit = jnp.sum(mw)
        tiles.append(jnp.sum(mw * jj) + (1.0 - hit) * (_NTT - 1))
        exps.append(jnp.sum(mw * ee.astype(jnp.float32)) + (1.0 - hit) * (_E - 1))
        sts.append(jnp.sum(mw * w_start))
        ens.append(jnp.sum(mw * w_end))
        typs.append(jnp.float32(0.0))
        outs.append(jnp.float32(0.0))
    for j in range(_NCT):
        tiles.append(jnp.float32(j))
        exps.append(jnp.float32(_E))
        sts.append(jnp.float32(0.0))
        ens.append(jnp.float32(0.0))
        typs.append(jnp.float32(1.0))
        outs.append(jnp.float32(j))
    ords = []
    r = jnp.float32(0.0)
    for w in range(_NG):
        if w > 0:
            r = r + (exps[w] != exps[w - 1]).astype(jnp.float32)
        ords.append(r)
    nexts = [None] * _NG
    nxt = exps[_NG - 1]
    for w in range(_NG - 1, -1, -1):
        nexts[w] = nxt
        if w > 0:
            nxt = jnp.where(exps[w] != exps[w - 1], exps[w], nxt)
    for w in range(_NG):
        p_ref[w] = tiles[w].astype(jnp.int32)
        p_ref[_S + w] = exps[w].astype(jnp.int32)
        p_ref[2 * _S + w] = sts[w].astype(jnp.int32)
        p_ref[3 * _S + w] = ens[w].astype(jnp.int32)
        p_ref[4 * _S + w] = ords[w].astype(jnp.int32)
        p_ref[5 * _S + w] = nexts[w].astype(jnp.int32)
        p_ref[6 * _S + w] = typs[w].astype(jnp.int32)
        p_ref[7 * _S + w] = outs[w].astype(jnp.int32)


def _mega_body(p_ref, xs_ref, x_ref,
               we1_ref, we2_ref, w1_ref, w2_ref,
               be1_ref, be2_ref, b1_ref, b2_ref,
               ys_ref, x1_ref, ring1, ring2, w1s, w2s, sems):
    w = pl.program_id(0)
    pidx = jnp.maximum(w - 1, 0)
    e = p_ref[_S + w]
    slot = lax.rem(p_ref[4 * _S + w], 2)
    new_exp = (w == 0) | (p_ref[_S + w] != p_ref[_S + pidx])
    typ = p_ref[6 * _S + w]

    def _start(eid, sl):
        @pl.when(eid <= _E - 1)
        def _():
            pltpu.make_async_copy(we1_ref.at[eid], ring1.at[sl],
                                  sems.at[0, sl]).start()
            pltpu.make_async_copy(we2_ref.at[eid], ring2.at[sl],
                                  sems.at[1, sl]).start()

        @pl.when(eid == _E)
        def _():
            pltpu.make_async_copy(w1_ref, ring1.at[sl], sems.at[0, sl]).start()
            pltpu.make_async_copy(w2_ref, ring2.at[sl], sems.at[1, sl]).start()

    @pl.when(w == 0)
    def _():
        _start(e, slot)

    @pl.when(new_exp)
    def _():
        @pl.when(e <= _E - 1)
        def _():
            pltpu.make_async_copy(we1_ref.at[e], ring1.at[slot],
                                  sems.at[0, slot]).wait()
            pltpu.make_async_copy(we2_ref.at[e], ring2.at[slot],
                                  sems.at[1, slot]).wait()

        @pl.when(e == _E)
        def _():
            pltpu.make_async_copy(w1_ref, ring1.at[slot], sems.at[0, slot]).wait()
            pltpu.make_async_copy(w2_ref, ring2.at[slot], sems.at[1, slot]).wait()

        w1s[...] = ring1[slot].astype(jnp.bfloat16)
        w2s[...] = ring2[slot].astype(jnp.bfloat16)
        nxt = p_ref[5 * _S + w]

        @pl.when(nxt != e)
        def _():
            _start(nxt, 1 - slot)

    tile = p_ref[w]

    @pl.when(typ == 0)
    def _():
        start = p_ref[2 * _S + w]
        end = p_ref[3 * _S + w]
        xt = xs_ref[...]
        h = jnp.dot(xt, w1s[...], preferred_element_type=jnp.float32)
        h = jnp.maximum(h + be1_ref[0], 0.0).astype(jnp.bfloat16)
        y = jnp.dot(h, w2s[...], preferred_element_type=jnp.float32) + be2_ref[0]
        sl = tile * _BT + lax.broadcasted_iota(jnp.int32, (_BT, 1), 0)
        msk = (sl >= start) & (sl < end)
        yw = jnp.where(msk, y, 0.0).astype(jnp.bfloat16)
        first = (w == 0) | (p_ref[w] != p_ref[pidx])

        @pl.when(first)
        def _():
            ys_ref[...] = yw

        @pl.when(jnp.logical_not(first))
        def _():
            ys_ref[...] = ys_ref[...] + yw

    @pl.when(typ == 1)
    def _():
        xb = x_ref[...].astype(jnp.bfloat16)
        h = jnp.dot(xb, w1s[...], preferred_element_type=jnp.float32)
        h = jnp.maximum(h + b1_ref[...], 0.0).astype(jnp.bfloat16)
        x1 = jnp.dot(h, w2s[...], preferred_element_type=jnp.float32) + b2_ref[...]
        x1_ref[...] = x1.astype(jnp.bfloat16)


def _fin_body(x_ref, x1_ref, ys_ref, d_ref, coef_ref, b0_ref,
              gamma_ref, beta_ref, o_ref):
    x = x_ref[...]
    dcol = d_ref[...]
    slots = lax.broadcasted_iota(jnp.int32, (1, _T), 1)
    b = (dcol == slots).astype(jnp.bfloat16)
    y2 = jnp.dot(b, ys_ref[...], preferred_element_type=jnp.float32)
    out = (b0_ref[...] * x1_ref[...].astype(jnp.float32)
           + coef_ref[...] * y2 + x)
    mu = jnp.mean(out, axis=1, keepdims=True)
    c = out - mu
    var = jnp.mean(c * c, axis=1, keepdims=True)
    o_ref[...] = c * lax.rsqrt(var + 1e-5) * gamma_ref[...] + beta_ref[...]


def kernel(x, W1, b1, W2, b2, Wg, bg, We1, be1, We2, be2, Wa, ba, gamma, beta):
    f32 = jnp.float32
    d, p, coef, b0, loss, xs = pl.pallas_call(
        _router_body,
        out_shape=[
            jax.ShapeDtypeStruct((_T, 1), jnp.int32),
            jax.ShapeDtypeStruct((512,), jnp.int32),
            jax.ShapeDtypeStruct((_T, 1), f32),
            jax.ShapeDtypeStruct((_T, 1), f32),
            jax.ShapeDtypeStruct((1, 1), f32),
            jax.ShapeDtypeStruct((_T, _D), jnp.bfloat16),
        ],
        out_specs=[
            pl.BlockSpec((_T, 1), lambda: (0, 0)),
            pl.BlockSpec(memory_space=pltpu.SMEM),
            pl.BlockSpec((_T, 1), lambda: (0, 0)),
            pl.BlockSpec((_T, 1), lambda: (0, 0)),
            pl.BlockSpec((1, 1), lambda: (0, 0)),
            pl.BlockSpec((_T, _D), lambda: (0, 0)),
        ],
    )(x, Wg, bg.reshape(1, _E), Wa, ba.reshape(1, 2))

    hbm = pltpu.MemorySpace.HBM
    grid_spec = pltpu.PrefetchScalarGridSpec(
        num_scalar_prefetch=1,
        grid=(_NG,),
        in_specs=[
            pl.BlockSpec((_BT, _D), lambda w, p: (p[w], 0)),
            pl.BlockSpec((_BTC, _D),
                         lambda w, p: (jnp.minimum(p[w], _NCT - 1), 0)),
            pl.BlockSpec(memory_space=hbm),
            pl.BlockSpec(memory_space=hbm),
            pl.BlockSpec(memory_space=hbm),
            pl.BlockSpec(memory_space=hbm),
            pl.BlockSpec((1, 1, _H),
                         lambda w, p: (jnp.minimum(p[_S + w], _E - 1), 0, 0)),
            pl.BlockSpec((1, 1, _D),
                         lambda w, p: (jnp.minimum(p[_S + w], _E - 1), 0, 0)),
            pl.BlockSpec((1, _H), lambda w, p: (0, 0)),
            pl.BlockSpec((1, _D), lambda w, p: (0, 0)),
        ],
        out_specs=[
            pl.BlockSpec((_BT, _D),
                         lambda w, p: (jnp.where(p[6 * _S + w] == 0,
                                                 p[w], _NTT - 1), 0)),
            pl.BlockSpec((_BTC, _D), lambda w, p: (p[7 * _S + w], 0)),
        ],
        scratch_shapes=[
            pltpu.VMEM((2, _D, _H), f32),
            pltpu.VMEM((2, _H, _D), f32),
            pltpu.VMEM((_D, _H), jnp.bfloat16),
            pltpu.VMEM((_H, _D), jnp.bfloat16),
            pltpu.SemaphoreType.DMA((2, 2)),
        ],
    )
    ys, x1 = pl.pallas_call(
        _mega_body,
        grid_spec=grid_spec,
        out_shape=[jax.ShapeDtypeStruct((_T, _D), jnp.bfloat16),
                   jax.ShapeDtypeStruct((_T, _D), jnp.bfloat16)],
    )(p, xs, x, We1, We2, W1, W2,
      be1.reshape(_E, 1, _H), be2.reshape(_E, 1, _D),
      b1.reshape(1, _H), b2.reshape(1, _D))

    out = pl.pallas_call(
        _fin_body,
        grid=(_NCT,),
        in_specs=[
            pl.BlockSpec((_BTC, _D), lambda i: (i, 0)),
            pl.BlockSpec((_BTC, _D), lambda i: (i, 0)),
            pl.BlockSpec((_T, _D), lambda i: (0, 0)),
            pl.BlockSpec((_BTC, 1), lambda i: (i, 0)),
            pl.BlockSpec((_BTC, 1), lambda i: (i, 0)),
            pl.BlockSpec((_BTC, 1), lambda i: (i, 0)),
            pl.BlockSpec((1, _D), lambda i: (0, 0)),
            pl.BlockSpec((1, _D), lambda i: (0, 0)),
        ],
        out_specs=pl.BlockSpec((_BTC, _D), lambda i: (i, 0)),
        out_shape=jax.ShapeDtypeStruct((_T, _D), f32),
    )(x, x1, ys, d, coef, b0, gamma.reshape(1, _D), beta.reshape(1, _D))

    return out, loss.reshape(())


# speedup vs baseline: 1.1438x; 1.1438x over previous
import jax
import jax.numpy as jnp
from jax import lax
from jax.experimental import pallas as pl
from jax.experimental.pallas import tpu as pltpu

_T, _D, _H, _E = 2048, 1024, 2048, 8
_BT = 256
_BM = 128
_NT = _T // _BT


def _router_body(x_ref, wg_ref, bg_ref, wa_ref, ba_ref,
                 xs_ref, d_ref, offs_ref, coef_ref, b0_ref, loss_ref):
    x = x_ref[...]
    logits = jnp.dot(x, wg_ref[...], preferred_element_type=jnp.float32)
    logits = logits + bg_ref[...]
    m = jnp.max(logits, axis=1, keepdims=True)
    ex = jnp.exp(logits - m)
    probs = ex / jnp.sum(ex, axis=1, keepdims=True)
    iota_e = lax.broadcasted_iota(jnp.int32, (_T, _E), 1)
    pmax = jnp.max(probs, axis=1, keepdims=True)
    idx = jnp.min(jnp.where(probs == pmax, iota_e, _E), axis=1, keepdims=True)
    disp = (iota_e == idx).astype(jnp.float32)
    gate = jnp.sum(probs * disp, axis=1, keepdims=True)

    rr = lax.broadcasted_iota(jnp.int32, (_T, _T), 0)
    cc = lax.broadcasted_iota(jnp.int32, (_T, _T), 1)
    ltri = (rr >= cc).astype(jnp.bfloat16)
    cum = jnp.dot(ltri, disp.astype(jnp.bfloat16),
                  preferred_element_type=jnp.float32)
    cnt = jnp.sum(disp, axis=0, keepdims=True)
    rank = jnp.sum(cum * disp, axis=1, keepdims=True) - 1.0
    eE_r = lax.broadcasted_iota(jnp.int32, (_E, _E), 0)
    eE_c = lax.broadcasted_iota(jnp.int32, (_E, _E), 1)
    excl = jnp.sum(jnp.transpose(cnt) * (eE_r < eE_c).astype(jnp.float32),
                   axis=0, keepdims=True)
    off_tok = jnp.sum(disp * excl, axis=1, keepdims=True)
    d = (off_tok + rank).astype(jnp.int32)
    d_ref[...] = d
    k16 = lax.broadcasted_iota(jnp.int32, (16, _E), 0)
    e16 = lax.broadcasted_iota(jnp.int32, (16, _E), 1)
    offs_ref[...] = jnp.sum(cnt * (e16 < k16).astype(jnp.float32),
                            axis=1, keepdims=True).astype(jnp.int32)

    sump = jnp.sum(probs, axis=0, keepdims=True)
    loss_ref[...] = (_E / (_T * _T)) * jnp.sum(cnt * sump, keepdims=True).reshape(1, 1)

    bl = jnp.dot(x, wa_ref[...], preferred_element_type=jnp.float32) + ba_ref[...]
    bm = jnp.max(bl, axis=1, keepdims=True)
    be = jnp.exp(bl - bm)
    bal = be / jnp.sum(be, axis=1, keepdims=True)
    b0_ref[...] = bal[:, 0:1]
    coef_ref[...] = gate * bal[:, 1:2]

    x_bf = x.astype(jnp.bfloat16)
    for j in range(_NT):
        s0 = j * _BT
        slot_ids = s0 + lax.broadcasted_iota(jnp.int32, (1, _BT), 1)
        a = (d == slot_ids).astype(jnp.bfloat16)
        xt = lax.dot_general(a, x_bf, (((0,), (0,)), ((), ())),
                             preferred_element_type=jnp.float32)
        xs_ref[pl.ds(s0, _BT), :] = xt.astype(jnp.bfloat16)


def _moe_body(offs_ref, xs_ref, we1_ref, we2_ref, be1_ref, be2_ref,
              ys_ref, w1_scr, w2_scr):
    e = pl.program_id(0)

    @pl.when(e == 0)
    def _():
        ys_ref[...] = jnp.zeros((_T, _D), jnp.bfloat16)

    w1_scr[...] = we1_ref[0].astype(jnp.bfloat16)
    w2_scr[...] = we2_ref[0].astype(jnp.bfloat16)
    start = offs_ref[e]
    end = offs_ref[e + 1]
    j0 = start // _BM
    j1 = (end + _BM - 1) // _BM
    b1v = be1_ref[0]
    b2v = be2_ref[0]

    def body(j, carry):
        s0 = pl.multiple_of(j * _BM, _BM)
        xt = xs_ref[pl.ds(s0, _BM), :]
        h = jnp.dot(xt, w1_scr[...], preferred_element_type=jnp.float32) + b1v
        h = jnp.maximum(h, 0.0).astype(jnp.bfloat16)
        y = jnp.dot(h, w2_scr[...], preferred_element_type=jnp.float32) + b2v
        sl = s0 + lax.broadcasted_iota(jnp.int32, (_BM, 1), 0)
        msk = (sl >= start) & (sl < end)
        yw = jnp.where(msk, y, 0.0).astype(jnp.bfloat16)
        ys_ref[pl.ds(s0, _BM), :] = ys_ref[pl.ds(s0, _BM), :] + yw
        return carry

    lax.fori_loop(j0, j1, body, 0)


def _out_body(x_ref, w1_ref, b1_ref, w2_ref, b2_ref, ys_ref, d_ref,
              coef_ref, b0_ref, gamma_ref, beta_ref, o_ref, w1s, w2s):
    i = pl.program_id(0)

    @pl.when(i == 0)
    def _():
        w1s[...] = w1_ref[...].astype(jnp.bfloat16)
        w2s[...] = w2_ref[...].astype(jnp.bfloat16)

    x = x_ref[...]
    xb = x.astype(jnp.bfloat16)
    h = jnp.dot(xb, w1s[...], preferred_element_type=jnp.float32) + b1_ref[...]
    h = jnp.maximum(h, 0.0).astype(jnp.bfloat16)
    x1 = jnp.dot(h, w2s[...], preferred_element_type=jnp.float32) + b2_ref[...]
    dcol = d_ref[...]
    slots = lax.broadcasted_iota(jnp.int32, (1, _T), 1)
    b = (dcol == slots).astype(jnp.bfloat16)
    y2 = jnp.dot(b, ys_ref[...], preferred_element_type=jnp.float32)
    out = b0_ref[...] * x1 + coef_ref[...] * y2 + x
    mu = jnp.mean(out, axis=1, keepdims=True)
    c = out - mu
    var = jnp.mean(c * c, axis=1, keepdims=True)
    o_ref[...] = c * lax.rsqrt(var + 1e-5) * gamma_ref[...] + beta_ref[...]


def kernel(x, W1, b1, W2, b2, Wg, bg, We1, be1, We2, be2, Wa, ba, gamma, beta):
    f32 = jnp.float32
    xs, d, offs, coef, b0, loss = pl.pallas_call(
        _router_body,
        out_shape=[
            jax.ShapeDtypeStruct((_T, _D), jnp.bfloat16),
            jax.ShapeDtypeStruct((_T, 1), jnp.int32),
            jax.ShapeDtypeStruct((16, 1), jnp.int32),
            jax.ShapeDtypeStruct((_T, 1), f32),
            jax.ShapeDtypeStruct((_T, 1), f32),
            jax.ShapeDtypeStruct((1, 1), f32),
        ],
    )(x, Wg, bg.reshape(1, _E), Wa, ba.reshape(1, 2))

    grid_spec = pltpu.PrefetchScalarGridSpec(
        num_scalar_prefetch=1,
        grid=(_E,),
        in_specs=[
            pl.BlockSpec((_T, _D), lambda e, offs: (0, 0)),
            pl.BlockSpec((1, _D, _H), lambda e, offs: (e, 0, 0)),
            pl.BlockSpec((1, _H, _D), lambda e, offs: (e, 0, 0)),
            pl.BlockSpec((1, 1, _H), lambda e, offs: (e, 0, 0)),
            pl.BlockSpec((1, 1, _D), lambda e, offs: (e, 0, 0)),
        ],
        out_specs=pl.BlockSpec((_T, _D), lambda e, offs: (0, 0)),
        scratch_shapes=[pltpu.VMEM((_D, _H), jnp.bfloat16),
                        pltpu.VMEM((_H, _D), jnp.bfloat16)],
    )
    ys = pl.pallas_call(
        _moe_body,
        grid_spec=grid_spec,
        out_shape=jax.ShapeDtypeStruct((_T, _D), jnp.bfloat16),
    )(offs.reshape(16), xs, We1, We2, be1.reshape(_E, 1, _H), be2.reshape(_E, 1, _D))

    out = pl.pallas_call(
        _out_body,
        grid=(_NT,),
        in_specs=[
            pl.BlockSpec((_BT, _D), lambda i: (i, 0)),
            pl.BlockSpec((_D, _H), lambda i: (0, 0)),
            pl.BlockSpec((1, _H), lambda i: (0, 0)),
            pl.BlockSpec((_H, _D), lambda i: (0, 0)),
            pl.BlockSpec((1, _D), lambda i: (0, 0)),
            pl.BlockSpec((_T, _D), lambda i: (0, 0)),
            pl.BlockSpec((_BT, 1), lambda i: (i, 0)),
            pl.BlockSpec((_BT, 1), lambda i: (i, 0)),
            pl.BlockSpec((_BT, 1), lambda i: (i, 0)),
            pl.BlockSpec((1, _D), lambda i: (0, 0)),
            pl.BlockSpec((1, _D), lambda i: (0, 0)),
        ],
        out_specs=pl.BlockSpec((_BT, _D), lambda i: (i, 0)),
        out_shape=jax.ShapeDtypeStruct((_T, _D), f32),
        scratch_shapes=[pltpu.VMEM((_D, _H), jnp.bfloat16),
                        pltpu.VMEM((_H, _D), jnp.bfloat16)],
    )(x, W1, b1.reshape(1, _H), W2, b2.reshape(1, _D), ys, d, coef, b0,
      gamma.reshape(1, _D), beta.reshape(1, _D))

    return out, loss.reshape(())


# speedup vs baseline: 1.1457x; 1.0016x over previous
import jax
import jax.numpy as jnp
from jax import lax
from jax.experimental import pallas as pl
from jax.experimental.pallas import tpu as pltpu

_T, _D, _H, _E = 2048, 1024, 2048, 8
_BT = 256
_BM = 128
_NT = _T // _BT


def _router_body(x_ref, wg_ref, bg_ref, wa_ref, ba_ref,
                 xs_ref, d_ref, offs_ref, coef_ref, b0_ref, loss_ref):
    x = x_ref[...]
    logits = jnp.dot(x, wg_ref[...], preferred_element_type=jnp.float32)
    logits = logits + bg_ref[...]
    m = jnp.max(logits, axis=1, keepdims=True)
    ex = jnp.exp(logits - m)
    probs = ex / jnp.sum(ex, axis=1, keepdims=True)
    iota_e = lax.broadcasted_iota(jnp.int32, (_T, _E), 1)
    pmax = jnp.max(probs, axis=1, keepdims=True)
    idx = jnp.min(jnp.where(probs == pmax, iota_e, _E), axis=1, keepdims=True)
    disp = (iota_e == idx).astype(jnp.float32)
    gate = jnp.sum(probs * disp, axis=1, keepdims=True)

    rr = lax.broadcasted_iota(jnp.int32, (_T, _T), 0)
    cc = lax.broadcasted_iota(jnp.int32, (_T, _T), 1)
    ltri = (rr >= cc).astype(jnp.bfloat16)
    cum = jnp.dot(ltri, disp.astype(jnp.bfloat16),
                  preferred_element_type=jnp.float32)
    cnt = jnp.sum(disp, axis=0, keepdims=True)
    rank = jnp.sum(cum * disp, axis=1, keepdims=True) - 1.0
    eE_r = lax.broadcasted_iota(jnp.int32, (_E, _E), 0)
    eE_c = lax.broadcasted_iota(jnp.int32, (_E, _E), 1)
    excl = jnp.sum(jnp.transpose(cnt) * (eE_r < eE_c).astype(jnp.float32),
                   axis=0, keepdims=True)
    off_tok = jnp.sum(disp * excl, axis=1, keepdims=True)
    d_f = off_tok + rank
    d_rowf = jnp.transpose(d_f)
    d = d_f.astype(jnp.int32)
    d_ref[...] = d
    k16 = lax.broadcasted_iota(jnp.int32, (16, _E), 0)
    e16 = lax.broadcasted_iota(jnp.int32, (16, _E), 1)
    offs_ref[...] = jnp.sum(cnt * (e16 < k16).astype(jnp.float32),
                            axis=1, keepdims=True).astype(jnp.int32)

    sump = jnp.sum(probs, axis=0, keepdims=True)
    loss_ref[...] = (_E / (_T * _T)) * jnp.sum(cnt * sump, keepdims=True).reshape(1, 1)

    bl = jnp.dot(x, wa_ref[...], preferred_element_type=jnp.float32) + ba_ref[...]
    bm = jnp.max(bl, axis=1, keepdims=True)
    be = jnp.exp(bl - bm)
    bal = be / jnp.sum(be, axis=1, keepdims=True)
    b0_ref[...] = bal[:, 0:1]
    coef_ref[...] = gate * bal[:, 1:2]

    x_bf = x.astype(jnp.bfloat16)
    for j in range(_NT):
        s0 = j * _BT
        slot_col = s0 + lax.broadcasted_iota(jnp.int32, (_BT, 1), 0)
        a2 = (slot_col.astype(jnp.float32) == d_rowf).astype(jnp.bfloat16)
        xt = jnp.dot(a2, x_bf, preferred_element_type=jnp.float32)
        xs_ref[pl.ds(s0, _BT), :] = xt.astype(jnp.bfloat16)


def _moe_body(offs_ref, xs_ref, we1_ref, we2_ref, be1_ref, be2_ref,
              ys_ref, w1_scr, w2_scr):
    e = pl.program_id(0)

    @pl.when(e == 0)
    def _():
        ys_ref[...] = jnp.zeros((_T, _D), jnp.bfloat16)

    w1_scr[...] = we1_ref[0].astype(jnp.bfloat16)
    w2_scr[...] = we2_ref[0].astype(jnp.bfloat16)
    start = offs_ref[e]
    end = offs_ref[e + 1]
    j0 = start // _BM
    j1 = (end + _BM - 1) // _BM
    b1v = be1_ref[0]
    b2v = be2_ref[0]

    def body(j, carry):
        s0 = pl.multiple_of(j * _BM, _BM)
        xt = xs_ref[pl.ds(s0, _BM), :]
        h = jnp.dot(xt, w1_scr[...], preferred_element_type=jnp.float32) + b1v
        h = jnp.maximum(h, 0.0).astype(jnp.bfloat16)
        y = jnp.dot(h, w2_scr[...], preferred_element_type=jnp.float32) + b2v
        sl = s0 + lax.broadcasted_iota(jnp.int32, (_BM, 1), 0)
        msk = (sl >= start) & (sl < end)
        yw = jnp.where(msk, y, 0.0).astype(jnp.bfloat16)
        ys_ref[pl.ds(s0, _BM), :] = ys_ref[pl.ds(s0, _BM), :] + yw
        return carry

    lax.fori_loop(j0, j1, body, 0)


def _out_body(x_ref, w1_ref, b1_ref, w2_ref, b2_ref, ys_ref, d_ref,
              coef_ref, b0_ref, gamma_ref, beta_ref, o_ref, w1s, w2s):
    i = pl.program_id(0)

    @pl.when(i == 0)
    def _():
        w1s[...] = w1_ref[...].astype(jnp.bfloat16)
        w2s[...] = w2_ref[...].astype(jnp.bfloat16)

    x = x_ref[...]
    xb = x.astype(jnp.bfloat16)
    h = jnp.dot(xb, w1s[...], preferred_element_type=jnp.float32) + b1_ref[...]
    h = jnp.maximum(h, 0.0).astype(jnp.bfloat16)
    x1 = jnp.dot(h, w2s[...], preferred_element_type=jnp.float32) + b2_ref[...]
    dcol = d_ref[...]
    slots = lax.broadcasted_iota(jnp.int32, (1, _T), 1)
    b = (dcol == slots).astype(jnp.bfloat16)
    y2 = jnp.dot(b, ys_ref[...], preferred_element_type=jnp.float32)
    out = b0_ref[...] * x1 + coef_ref[...] * y2 + x
    mu = jnp.mean(out, axis=1, keepdims=True)
    c = out - mu
    var = jnp.mean(c * c, axis=1, keepdims=True)
    o_ref[...] = c * lax.rsqrt(var + 1e-5) * gamma_ref[...] + beta_ref[...]


def kernel(x, W1, b1, W2, b2, Wg, bg, We1, be1, We2, be2, Wa, ba, gamma, beta):
    f32 = jnp.float32
    xs, d, offs, coef, b0, loss = pl.pallas_call(
        _router_body,
        out_shape=[
            jax.ShapeDtypeStruct((_T, _D), jnp.bfloat16),
            jax.ShapeDtypeStruct((_T, 1), jnp.int32),
            jax.ShapeDtypeStruct((16, 1), jnp.int32),
            jax.ShapeDtypeStruct((_T, 1), f32),
            jax.ShapeDtypeStruct((_T, 1), f32),
            jax.ShapeDtypeStruct((1, 1), f32),
        ],
    )(x, Wg, bg.reshape(1, _E), Wa, ba.reshape(1, 2))

    grid_spec = pltpu.PrefetchScalarGridSpec(
        num_scalar_prefetch=1,
        grid=(_E,),
        in_specs=[
            pl.BlockSpec((_T, _D), lambda e, offs: (0, 0)),
            pl.BlockSpec((1, _D, _H), lambda e, offs: (e, 0, 0)),
            pl.BlockSpec((1, _H, _D), lambda e, offs: (e, 0, 0)),
            pl.BlockSpec((1, 1, _H), lambda e, offs: (e, 0, 0)),
            pl.BlockSpec((1, 1, _D), lambda e, offs: (e, 0, 0)),
        ],
        out_specs=pl.BlockSpec((_T, _D), lambda e, offs: (0, 0)),
        scratch_shapes=[pltpu.VMEM((_D, _H), jnp.bfloat16),
                        pltpu.VMEM((_H, _D), jnp.bfloat16)],
    )
    ys = pl.pallas_call(
        _moe_body,
        grid_spec=grid_spec,
        out_shape=jax.ShapeDtypeStruct((_T, _D), jnp.bfloat16),
    )(offs.reshape(16), xs, We1, We2, be1.reshape(_E, 1, _H), be2.reshape(_E, 1, _D))

    out = pl.pallas_call(
        _out_body,
        grid=(_NT,),
        in_specs=[
            pl.BlockSpec((_BT, _D), lambda i: (i, 0)),
            pl.BlockSpec((_D, _H), lambda i: (0, 0)),
            pl.BlockSpec((1, _H), lambda i: (0, 0)),
            pl.BlockSpec((_H, _D), lambda i: (0, 0)),
            pl.BlockSpec((1, _D), lambda i: (0, 0)),
            pl.BlockSpec((_T, _D), lambda i: (0, 0)),
            pl.BlockSpec((_BT, 1), lambda i: (i, 0)),
            pl.BlockSpec((_BT, 1), lambda i: (i, 0)),
            pl.BlockSpec((_BT, 1), lambda i: (i, 0)),
            pl.BlockSpec((1, _D), lambda i: (0, 0)),
            pl.BlockSpec((1, _D), lambda i: (0, 0)),
        ],
        out_specs=pl.BlockSpec((_BT, _D), lambda i: (i, 0)),
        out_shape=jax.ShapeDtypeStruct((_T, _D), f32),
        scratch_shapes=[pltpu.VMEM((_D, _H), jnp.bfloat16),
                        pltpu.VMEM((_H, _D), jnp.bfloat16)],
    )(x, W1, b1.reshape(1, _H), W2, b2.reshape(1, _D), ys, d, coef, b0,
      gamma.reshape(1, _D), beta.reshape(1, _D))

    return out, loss.reshape(())
